# baseline (speedup 1.0000x reference)
import functools

import jax
import jax.numpy as jnp
from jax import lax
from jax.experimental import pallas as pl
from jax.experimental.pallas import tpu as pltpu
from jax.experimental.pallas import tpu_sc as plsc

_N = 10000
_E = 320000
_IN = 128
_HID = 256
_NEG = -1e30

_NODE_TILE = 1000
_EDGE_TILE = 3200
_HEAD_TILES = 4


_SC_NW = 32
_GCHUNK = 200


_GCHUNK = 400
_GSUPER = 5


def _sc_gather_body(x_hbm, idx_hbm, out_hbm,
                    idx_v, r0_, r1_, gsem0, gsem1, wsem0, wsem1):
    c = lax.axis_index("c")
    s = lax.axis_index("s")
    wid = s * 2 + c
    per_w = 2 * _E // _SC_NW
    base = wid * per_w
    sup = _GSUPER * _GCHUNK

    rbufs = (r0_, r1_)
    gsems = (gsem0, gsem1)
    wsems = (wsem0, wsem1)

    def super_body(j, carry):
        soff = base + j * sup
        pltpu.sync_copy(idx_hbm.at[pl.ds(soff, sup)], idx_v)

        def fire(r, b):
            sl = pl.ds(r * _GCHUNK, _GCHUNK)
            return pltpu.async_copy(x_hbm.at[idx_v.at[sl]], rbufs[b],
                                    gsems[b])

        pend_w = [None, None]
        pend = fire(0, 0)
        for r in range(_GSUPER):
            b = r & 1
            pend.wait()
            if r < _GSUPER - 1:
                if pend_w[1 - b] is not None:
                    pend_w[1 - b].wait()
                pend = fire(r + 1, 1 - b)
            off = soff + r * _GCHUNK
            pend_w[b] = pltpu.async_copy(
                rbufs[b], out_hbm.at[pl.ds(off, _GCHUNK)], wsems[b])
        for b in range(2):
            if pend_w[b] is not None:
                pend_w[b].wait()
        return carry

    lax.fori_loop(0, per_w // sup, super_body, 0)


def _sc_gather(x, idx_cat):
    mesh = plsc.VectorSubcoreMesh(core_axis_name="c", subcore_axis_name="s")
    fn = pl.kernel(
        _sc_gather_body,
        out_type=jax.ShapeDtypeStruct((2 * _E, _IN), jnp.float32),
        mesh=mesh,
        scratch_types=[
            pltpu.VMEM((_GSUPER * _GCHUNK,), jnp.int32),
            pltpu.VMEM((_GCHUNK, _IN), jnp.float32),
            pltpu.VMEM((_GCHUNK, _IN), jnp.float32),
            pltpu.SemaphoreType.DMA,
            pltpu.SemaphoreType.DMA,
            pltpu.SemaphoreType.DMA,
            pltpu.SemaphoreType.DMA,
        ],
    )
    out = fn(x, idx_cat)
    return out[:_E], out[_E:]


_SCHUNK = 160
_ROWS_PER_SUB = 624
_ROW_PIECES = [(0, 160), (160, 160), (320, 160), (480, 144)]


def _relu_table_kernel(h_ref, emb_ref, out_ref):
    out_ref[...] = jax.nn.relu(h_ref[...] + emb_ref[0])[None]


def _relu_table(h_split, emb_split):
    out = pl.pallas_call(
        _relu_table_kernel,
        grid=(24,),
        in_specs=[
            pl.BlockSpec((_N, 128), lambda i: (i // 12, 0)),
            pl.BlockSpec((1, 1, 128), lambda i: (16 * (i // 12) + i % 12, 0, 0)),
        ],
        out_specs=pl.BlockSpec((1, _N, 128), lambda i: (i, 0, 0)),
        out_shape=jax.ShapeDtypeStruct((24, _N, 128), jnp.float32),
        compiler_params=_ARB,
    )(h_split, emb_split[:, None, :])
    return jnp.reshape(out, (24 * _N, 128))


_SSUPER = 25


def _sc_scatter_body(tab_hbm, src_hbm, dst_hbm, combo_hbm, out_hbm,
                     idx_src, dst0, dst1, idx_cmb, rows0, rows1, agg,
                     gsem0, gsem1, dsem0, dsem1, ssem0, ssem1):
    c = lax.axis_index("c")
    s = lax.axis_index("s")
    per_s = _E // 16
    base = s * per_s
    sup = _SSUPER * _SCHUNK

    def zero_buf(e, carry):
        for g in range(_IN // 16):
            rows0[e, pl.ds(g * 16, 16)] = jnp.zeros((16,), jnp.float32)
        return carry
    lax.fori_loop(0, _SCHUNK, zero_buf, 0)
    r0 = s * _ROWS_PER_SUB
    for (o, nrow) in _ROW_PIECES:
        pltpu.sync_copy(rows0.at[pl.ds(0, nrow)],
                        agg.at[pl.ds(r0 + o, nrow)])

    @pl.when(s == 15)
    def _():
        pltpu.sync_copy(rows0.at[pl.ds(0, _N - 16 * _ROWS_PER_SUB)],
                        agg.at[pl.ds(16 * _ROWS_PER_SUB,
                                     _N - 16 * _ROWS_PER_SUB)])
    plsc.subcore_barrier()

    rbufs = (rows0, rows1)
    dbufs = (dst0, dst1)
    gsems = (gsem0, gsem1)
    dsems = (dsem0, dsem1)
    ssems = (ssem0, ssem1)

    def super_body(j, carry):
        soff = base + j * sup
        pltpu.sync_copy(src_hbm.at[pl.ds(soff, sup)], idx_src)
        pltpu.sync_copy(combo_hbm.at[pl.ds(soff, sup)], idx_cmb)

        def gidx(k, carry2):
            sl = pl.ds(k * 16, 16)
            idx_src[sl] = (idx_src[sl] + idx_cmb[sl] * _N
                           + jnp.full((16,), c * 12 * _N, jnp.int32))
            return carry2
        lax.fori_loop(0, sup // 16, gidx, 0, unroll=5)

        def fire(r, b):
            sl = pl.ds(r * _SCHUNK, _SCHUNK)
            g = pltpu.async_copy(tab_hbm.at[idx_src.at[sl]], rbufs[b],
                                 gsems[b])
            d = pltpu.async_copy(dst_hbm.at[pl.ds(soff + r * _SCHUNK,
                                                  _SCHUNK)],
                                 dbufs[b], dsems[b])
            return (g, d)

        pend_s = [None, None]
        pend = fire(0, 0)
        for r in range(_SSUPER):
            b = r & 1
            pend[0].wait()
            pend[1].wait()
            if r < _SSUPER - 1:
                if pend_s[1 - b] is not None:
                    pend_s[1 - b].wait()
                pend = fire(r + 1, 1 - b)
            pend_s[b] = pltpu.async_copy(rbufs[b], agg.at[dbufs[b]],
                                         ssems[b], add=True)
        for b in range(2):
            if pend_s[b] is not None:
                pend_s[b].wait()
        return carry

    lax.fori_loop(0, per_s // sup, super_body, 0)
    plsc.subcore_barrier()

    for (o, nrow) in _ROW_PIECES:
        pltpu.sync_copy(agg.at[pl.ds(r0 + o, nrow)],
                        rows0.at[pl.ds(0, nrow)])
        pltpu.sync_copy(rows0.at[pl.ds(0, nrow)],
                        out_hbm.at[c, pl.ds(r0 + o, nrow)])

    @pl.when(s == 15)
    def _():
        tail = _N - 16 * _ROWS_PER_SUB
        pltpu.sync_copy(agg.at[pl.ds(16 * _ROWS_PER_SUB, tail)],
                        rows0.at[pl.ds(0, tail)])
        pltpu.sync_copy(rows0.at[pl.ds(0, tail)],
                        out_hbm.at[c, pl.ds(16 * _ROWS_PER_SUB, tail)])


def _sc_scatter(h, emb12, src, dst, combo):
    h_split = jnp.concatenate([h[:, :128], h[:, 128:]], axis=0)
    emb_split = jnp.zeros((32, 128), jnp.float32)
    emb_split = emb_split.at[0:12].set(emb12[:, :128])
    emb_split = emb_split.at[16:28].set(emb12[:, 128:])
    tab = _relu_table(h_split, emb_split)
    mesh = plsc.VectorSubcoreMesh(core_axis_name="c", subcore_axis_name="s")
    fn = pl.kernel(
        _sc_scatter_body,
        out_type=jax.ShapeDtypeStruct((2, _N, 128), jnp.float32),
        mesh=mesh,
        scratch_types=[
            pltpu.VMEM((_SSUPER * _SCHUNK,), jnp.int32),
            pltpu.VMEM((_SCHUNK,), jnp.int32),
            pltpu.VMEM((_SCHUNK,), jnp.int32),
            pltpu.VMEM((_SSUPER * _SCHUNK,), jnp.int32),
            pltpu.VMEM((_SCHUNK, 128), jnp.float32),
            pltpu.VMEM((_SCHUNK, 128), jnp.float32),
            pltpu.VMEM_SHARED((_N, 128), jnp.float32),
            pltpu.SemaphoreType.DMA,
            pltpu.SemaphoreType.DMA,
            pltpu.SemaphoreType.DMA,
            pltpu.SemaphoreType.DMA,
            pltpu.SemaphoreType.DMA,
            pltpu.SemaphoreType.DMA,
        ],
    )
    out = fn(tab, src, dst, combo)
    return jnp.concatenate([out[0], out[1]], axis=1)


def _enc2_kernel(x_ref, wa_ref, ba_ref, wc_ref, bc_ref, ha_ref, hc_ref):
    r = jax.nn.relu(x_ref[...])
    ha_ref[...] = jnp.dot(r, wa_ref[...], preferred_element_type=jnp.float32) + ba_ref[...]
    hc_ref[...] = jnp.dot(r, wc_ref[...], preferred_element_type=jnp.float32) + bc_ref[...]


def _enc1_kernel(gs_ref, gd_ref, w_ref, b_ref, h_ref):
    r = jax.nn.relu(gs_ref[...].astype(jnp.float32)
                    + gd_ref[...].astype(jnp.float32))
    h_ref[...] = jnp.dot(r, w_ref[...], preferred_element_type=jnp.float32) + b_ref[...]


def _ce_block(logits, lab_col, need_acc):
    m = jnp.max(logits, axis=1, keepdims=True)
    lse = m[:, 0] + jnp.log(jnp.sum(jnp.exp(logits - m), axis=1))
    iota = jax.lax.broadcasted_iota(jnp.int32, logits.shape, 1)
    onehot = iota == lab_col
    picked = jnp.sum(jnp.where(onehot, logits, 0.0), axis=1)
    loss_sum = jnp.sum(lse - picked)
    if need_acc:
        idx = jnp.min(jnp.where(logits == m, iota, jnp.int32(1 << 30)), axis=1)
        acc_sum = jnp.sum((idx == lab_col[:, 0]).astype(jnp.float32))
    else:
        acc_sum = jnp.float32(0.0)
    return loss_sum, acc_sum


def _node_loss_kernel(ha_ref, agga_ref, laba_ref, hc_ref, aggc_ref, labc_ref,
                      wa_ref, ba_ref, wc_ref, bc_ref, epsa_ref, epsc_ref,
                      out_ref):
    i = pl.program_id(0)

    @pl.when(i == 0)
    def _():
        out_ref[...] = jnp.zeros_like(out_ref)

    def one(h, agg, lab, w, bpad, eps, need_acc):
        z = h + eps * h + agg
        logits = jnp.dot(z, w, preferred_element_type=jnp.float32) + bpad
        return _ce_block(logits, lab, need_acc)

    la, acca = one(ha_ref[...], agga_ref[...], laba_ref[...],
                   wa_ref[...], ba_ref[...], epsa_ref[...], True)
    lc, _ = one(hc_ref[...], aggc_ref[...], labc_ref[...],
                wc_ref[...], bc_ref[...], epsc_ref[...], False)
    lane = jax.lax.broadcasted_iota(jnp.int32, (1, 128), 1)
    out_ref[...] += jnp.where(lane == 0, la,
                     jnp.where(lane == 1, lc,
                      jnp.where(lane == 2, acca, 0.0)))


def _edge_head_loss_kernel(gs_ref, gd_ref, lab_ref, agg_ref, we_ref, be_ref,
                           wo_ref, bo_ref, eps_ref, out_ref):
    i = pl.program_id(0)

    @pl.when(i == 0)
    def _():
        out_ref[...] = jnp.zeros_like(out_ref)

    h = jnp.dot(jax.nn.relu(gs_ref[...].astype(jnp.float32)
                            + gd_ref[...].astype(jnp.float32)), we_ref[...],
                preferred_element_type=jnp.float32) + be_ref[...]
    z = h + eps_ref[...] * h + agg_ref[...]
    logits = jnp.dot(z, wo_ref[...], preferred_element_type=jnp.float32) + bo_ref[...]
    ls, _ = _ce_block(logits, lab_ref[...], False)
    lane = jax.lax.broadcasted_iota(jnp.int32, (1, 128), 1)
    out_ref[...] += jnp.where(lane == 0, ls, 0.0)


def _edge_tail_loss_kernel(gs_ref, gd_ref, lab_ref, we_ref, be_ref,
                           wo_ref, bo_ref, eps_ref, out_ref):
    i = pl.program_id(0)

    @pl.when(i == 0)
    def _():
        out_ref[...] = jnp.zeros_like(out_ref)

    h = jnp.dot(jax.nn.relu(gs_ref[...].astype(jnp.float32)
                            + gd_ref[...].astype(jnp.float32)), we_ref[...],
                preferred_element_type=jnp.float32) + be_ref[...]
    z = h + eps_ref[...] * h
    logits = jnp.dot(z, wo_ref[...], preferred_element_type=jnp.float32) + bo_ref[...]
    ls, _ = _ce_block(logits, lab_ref[...], False)
    lane = jax.lax.broadcasted_iota(jnp.int32, (1, 128), 1)
    out_ref[...] += jnp.where(lane == 0, ls, 0.0)


def _pad_out_params(w, b, out_dim):
    wp = jnp.zeros((_HID, 128), jnp.float32).at[:, :out_dim].set(w)
    bp = jnp.full((1, 128), _NEG, jnp.float32).at[0, :out_dim].set(b)
    return wp, bp


def _combo_table(p):
    return (p["E_type"][:, None, :] + p["E_dire"][None, :, :]).reshape(-1, _HID)


_ARB = pltpu.CompilerParams(dimension_semantics=("arbitrary",))


def _run_node_loss(ha, agga, laba, hc, aggc, labc, pa, pc):
    wa, ba = _pad_out_params(pa["W_out"], pa["b_out"], 119)
    wc, bc = _pad_out_params(pc["W_out"], pc["b_out"], 4)
    n_tiles = _N // _NODE_TILE
    grid = (n_tiles,)
    tile = _NODE_TILE
    out = pl.pallas_call(
        _node_loss_kernel,
        grid=grid,
        in_specs=[
            pl.BlockSpec((tile, _HID), lambda i: (i, 0)),
            pl.BlockSpec((tile, _HID), lambda i: (i, 0)),
            pl.BlockSpec((tile, 1), lambda i: (i, 0)),
            pl.BlockSpec((tile, _HID), lambda i: (i, 0)),
            pl.BlockSpec((tile, _HID), lambda i: (i, 0)),
            pl.BlockSpec((tile, 1), lambda i: (i, 0)),
            pl.BlockSpec((_HID, 128), lambda i: (0, 0)),
            pl.BlockSpec((1, 128), lambda i: (0, 0)),
            pl.BlockSpec((_HID, 128), lambda i: (0, 0)),
            pl.BlockSpec((1, 128), lambda i: (0, 0)),
            pl.BlockSpec((1, 1), lambda i: (0, 0)),
            pl.BlockSpec((1, 1), lambda i: (0, 0)),
        ],
        out_specs=pl.BlockSpec((1, 128), lambda i: (0, 0)),
        out_shape=jax.ShapeDtypeStruct((1, 128), jnp.float32),
        compiler_params=_ARB,
    )(ha, agga, laba, hc, aggc, labc, wa, ba, wc, bc,
      jnp.reshape(pa["eps"], (1, 1)), jnp.reshape(pc["eps"], (1, 1)))
    return out[0, 0], out[0, 1], out[0, 2]


def _run_edge_loss(gs, gd, labe, agg_e, pe):
    wo, bo = _pad_out_params(pe["W_out"], pe["b_out"], 4)
    we = pe["W_enc"]
    be = jnp.reshape(pe["b_enc"], (1, _HID))
    eps = jnp.reshape(pe["eps"], (1, 1))
    head_n = _HEAD_TILES * _EDGE_TILE
    agg_pad = jnp.zeros((head_n, _HID), jnp.float32).at[:_N].set(agg_e)

    head = pl.pallas_call(
        _edge_head_loss_kernel,
        grid=(_HEAD_TILES,),
        in_specs=[
            pl.BlockSpec((_EDGE_TILE, _IN), lambda i: (i, 0)),
            pl.BlockSpec((_EDGE_TILE, _IN), lambda i: (i, 0)),
            pl.BlockSpec((_EDGE_TILE, 1), lambda i: (i, 0)),
            pl.BlockSpec((_EDGE_TILE, _HID), lambda i: (i, 0)),
            pl.BlockSpec((_IN, _HID), lambda i: (0, 0)),
            pl.BlockSpec((1, _HID), lambda i: (0, 0)),
            pl.BlockSpec((_HID, 128), lambda i: (0, 0)),
            pl.BlockSpec((1, 128), lambda i: (0, 0)),
            pl.BlockSpec((1, 1), lambda i: (0, 0)),
        ],
        out_specs=pl.BlockSpec((1, 128), lambda i: (0, 0)),
        out_shape=jax.ShapeDtypeStruct((1, 128), jnp.float32),
        compiler_params=_ARB,
    )(gs[:head_n], gd[:head_n], labe[:head_n], agg_pad, we, be, wo, bo, eps)

    tail = pl.pallas_call(
        _edge_tail_loss_kernel,
        grid=((_E - head_n) // _EDGE_TILE,),
        in_specs=[
            pl.BlockSpec((_EDGE_TILE, _IN), lambda i: (i, 0)),
            pl.BlockSpec((_EDGE_TILE, _IN), lambda i: (i, 0)),
            pl.BlockSpec((_EDGE_TILE, 1), lambda i: (i, 0)),
            pl.BlockSpec((_IN, _HID), lambda i: (0, 0)),
            pl.BlockSpec((1, _HID), lambda i: (0, 0)),
            pl.BlockSpec((_HID, 128), lambda i: (0, 0)),
            pl.BlockSpec((1, 128), lambda i: (0, 0)),
            pl.BlockSpec((1, 1), lambda i: (0, 0)),
        ],
        out_specs=pl.BlockSpec((1, 128), lambda i: (0, 0)),
        out_shape=jax.ShapeDtypeStruct((1, 128), jnp.float32),
        compiler_params=_ARB,
    )(gs[head_n:], gd[head_n:], labe[head_n:], we, be, wo, bo, eps)
    return head[0, 0] + tail[0, 0]


def kernel(node_representation, node_type, node_chiral_type, edge_type,
           edge_dire_type, edge_index, params):
    x = node_representation
    pa, pc, pe = params["atom"], params["chiral"], params["edge"]
    src = edge_index[0]
    dst = edge_index[1]
    combo = edge_type * 3 + edge_dire_type

    ha, hc = pl.pallas_call(
        _enc2_kernel,
        out_shape=[jax.ShapeDtypeStruct((_N, _HID), jnp.float32)] * 2,
    )(x, pa["W_enc"], jnp.reshape(pa["b_enc"], (1, _HID)),
      pc["W_enc"], jnp.reshape(pc["b_enc"], (1, _HID)))

    gs, gd = _sc_gather(x, jnp.concatenate([src, dst]))

    h_head = pl.pallas_call(
        _enc1_kernel,
        out_shape=jax.ShapeDtypeStruct((_N, _HID), jnp.float32),
    )(gs[:_N], gd[:_N], pe["W_enc"], jnp.reshape(pe["b_enc"], (1, _HID)))

    emb_a = _combo_table(pa)
    emb_c = _combo_table(pc)
    emb_e = _combo_table(pe)
    agg_a = _sc_scatter(ha, emb_a, src, dst, combo)
    agg_c = _sc_scatter(hc, emb_c, src, dst, combo)
    agg_e = _sc_scatter(h_head, emb_e, src, dst, combo)

    laba = node_type[:, None].astype(jnp.int32)
    labc = node_chiral_type[:, None].astype(jnp.int32)
    labe = edge_type[:, None].astype(jnp.int32)

    la_sum, lc_sum, acc_sum = _run_node_loss(ha, agg_a, laba, hc, agg_c, labc,
                                             pa, pc)
    le_sum = _run_edge_loss(gs, gd, labe, agg_e, pe)

    loss = la_sum / _N + lc_sum / _N + le_sum / _E
    acc = acc_sum / _N
    return (loss, acc)


# speedup vs baseline: 1.1981x; 1.1981x over previous
import functools

import jax
import jax.numpy as jnp
from jax import lax
from jax.experimental import pallas as pl
from jax.experimental.pallas import tpu as pltpu
from jax.experimental.pallas import tpu_sc as plsc

_N = 10000
_E = 320000
_IN = 128
_HID = 256
_NEG = -1e30

_NODE_TILE = 1000
_EDGE_TILE = 3200
_HEAD_TILES = 4


_SC_NW = 32
_GCHUNK = 200


_GCHUNK = 400
_GSUPER = 5


def _sc_gather_body(x_hbm, idx_hbm, out_hbm,
                    idx_v, r0_, r1_, gsem0, gsem1, wsem0, wsem1):
    c = lax.axis_index("c")
    s = lax.axis_index("s")
    wid = s * 2 + c
    per_w = 2 * _E // _SC_NW
    base = wid * per_w
    sup = _GSUPER * _GCHUNK

    rbufs = (r0_, r1_)
    gsems = (gsem0, gsem1)
    wsems = (wsem0, wsem1)

    def super_body(j, carry):
        soff = base + j * sup
        pltpu.sync_copy(idx_hbm.at[pl.ds(soff, sup)], idx_v)

        def fire(r, b):
            sl = pl.ds(r * _GCHUNK, _GCHUNK)
            return pltpu.async_copy(x_hbm.at[idx_v.at[sl]], rbufs[b],
                                    gsems[b])

        pend_w = [None, None]
        pend = fire(0, 0)
        for r in range(_GSUPER):
            b = r & 1
            pend.wait()
            if r < _GSUPER - 1:
                if pend_w[1 - b] is not None:
                    pend_w[1 - b].wait()
                pend = fire(r + 1, 1 - b)
            off = soff + r * _GCHUNK
            pend_w[b] = pltpu.async_copy(
                rbufs[b], out_hbm.at[pl.ds(off, _GCHUNK)], wsems[b])
        for b in range(2):
            if pend_w[b] is not None:
                pend_w[b].wait()
        return carry

    lax.fori_loop(0, per_w // sup, super_body, 0)


def _sc_gather(x, idx_cat):
    mesh = plsc.VectorSubcoreMesh(core_axis_name="c", subcore_axis_name="s")
    fn = pl.kernel(
        _sc_gather_body,
        out_type=jax.ShapeDtypeStruct((2 * _E, _IN), jnp.float32),
        mesh=mesh,
        scratch_types=[
            pltpu.VMEM((_GSUPER * _GCHUNK,), jnp.int32),
            pltpu.VMEM((_GCHUNK, _IN), jnp.float32),
            pltpu.VMEM((_GCHUNK, _IN), jnp.float32),
            pltpu.SemaphoreType.DMA,
            pltpu.SemaphoreType.DMA,
            pltpu.SemaphoreType.DMA,
            pltpu.SemaphoreType.DMA,
        ],
    )
    return fn(x, idx_cat)


_SCHUNK = 160
_ROWS_PER_SUB = 624
_ROW_PIECES = [(0, 160), (160, 160), (320, 160), (480, 144)]


def _relu_table_kernel(h_ref, emb_ref, out_ref):
    out_ref[...] = jax.nn.relu(h_ref[...] + emb_ref[0])[None]


def _relu_table(h_split, emb_split):
    out = pl.pallas_call(
        _relu_table_kernel,
        grid=(24,),
        in_specs=[
            pl.BlockSpec((_N, 128), lambda i: (i // 12, 0)),
            pl.BlockSpec((1, 1, 128), lambda i: (16 * (i // 12) + i % 12, 0, 0)),
        ],
        out_specs=pl.BlockSpec((1, _N, 128), lambda i: (i, 0, 0)),
        out_shape=jax.ShapeDtypeStruct((24, _N, 128), jnp.float32),
        compiler_params=_ARB,
    )(h_split, emb_split[:, None, :])
    return jnp.reshape(out, (24 * _N, 128))


_SSUPER = 25


def _sc_scatter_body(tab_hbm, src_hbm, dst_hbm, combo_hbm, out_hbm,
                     idx_src, dst0, dst1, idx_cmb, rows0, rows1, agg,
                     gsem0, gsem1, dsem0, dsem1, ssem0, ssem1):
    c = lax.axis_index("c")
    s = lax.axis_index("s")
    per_s = _E // 16
    base = s * per_s
    sup = _SSUPER * _SCHUNK

    def zero_buf(e, carry):
        for g in range(_IN // 16):
            rows0[e, pl.ds(g * 16, 16)] = jnp.zeros((16,), jnp.float32)
        return carry
    lax.fori_loop(0, _SCHUNK, zero_buf, 0)
    r0 = s * _ROWS_PER_SUB
    for (o, nrow) in _ROW_PIECES:
        pltpu.sync_copy(rows0.at[pl.ds(0, nrow)],
                        agg.at[pl.ds(r0 + o, nrow)])

    @pl.when(s == 15)
    def _():
        pltpu.sync_copy(rows0.at[pl.ds(0, _N - 16 * _ROWS_PER_SUB)],
                        agg.at[pl.ds(16 * _ROWS_PER_SUB,
                                     _N - 16 * _ROWS_PER_SUB)])
    plsc.subcore_barrier()

    rbufs = (rows0, rows1)
    dbufs = (dst0, dst1)
    gsems = (gsem0, gsem1)
    dsems = (dsem0, dsem1)
    ssems = (ssem0, ssem1)

    def super_body(j, carry):
        soff = base + j * sup
        pltpu.sync_copy(src_hbm.at[pl.ds(soff, sup)], idx_src)
        pltpu.sync_copy(combo_hbm.at[pl.ds(soff, sup)], idx_cmb)

        def gidx(k, carry2):
            sl = pl.ds(k * 16, 16)
            idx_src[sl] = (idx_src[sl] + idx_cmb[sl] * _N
                           + jnp.full((16,), c * 12 * _N, jnp.int32))
            return carry2
        lax.fori_loop(0, sup // 16, gidx, 0, unroll=5)

        def fire(r, b):
            sl = pl.ds(r * _SCHUNK, _SCHUNK)
            g = pltpu.async_copy(tab_hbm.at[idx_src.at[sl]], rbufs[b],
                                 gsems[b])
            d = pltpu.async_copy(dst_hbm.at[pl.ds(soff + r * _SCHUNK,
                                                  _SCHUNK)],
                                 dbufs[b], dsems[b])
            return (g, d)

        pend_s = [None, None]
        pend = fire(0, 0)
        for r in range(_SSUPER):
            b = r & 1
            pend[0].wait()
            pend[1].wait()
            if r < _SSUPER - 1:
                if pend_s[1 - b] is not None:
                    pend_s[1 - b].wait()
                pend = fire(r + 1, 1 - b)
            pend_s[b] = pltpu.async_copy(rbufs[b], agg.at[dbufs[b]],
                                         ssems[b], add=True)
        for b in range(2):
            if pend_s[b] is not None:
                pend_s[b].wait()
        return carry

    lax.fori_loop(0, per_s // sup, super_body, 0)
    plsc.subcore_barrier()

    for (o, nrow) in _ROW_PIECES:
        pltpu.sync_copy(agg.at[pl.ds(r0 + o, nrow)],
                        rows0.at[pl.ds(0, nrow)])
        pltpu.sync_copy(rows0.at[pl.ds(0, nrow)],
                        out_hbm.at[c, pl.ds(r0 + o, nrow)])

    @pl.when(s == 15)
    def _():
        tail = _N - 16 * _ROWS_PER_SUB
        pltpu.sync_copy(agg.at[pl.ds(16 * _ROWS_PER_SUB, tail)],
                        rows0.at[pl.ds(0, tail)])
        pltpu.sync_copy(rows0.at[pl.ds(0, tail)],
                        out_hbm.at[c, pl.ds(16 * _ROWS_PER_SUB, tail)])


def _sc_scatter(h, emb12, src, dst, combo):
    h_split = jnp.concatenate([h[:, :128], h[:, 128:]], axis=0)
    emb_split = jnp.zeros((32, 128), jnp.float32)
    emb_split = emb_split.at[0:12].set(emb12[:, :128])
    emb_split = emb_split.at[16:28].set(emb12[:, 128:])
    tab = _relu_table(h_split, emb_split)
    mesh = plsc.VectorSubcoreMesh(core_axis_name="c", subcore_axis_name="s")
    fn = pl.kernel(
        _sc_scatter_body,
        out_type=jax.ShapeDtypeStruct((2, _N, 128), jnp.float32),
        mesh=mesh,
        scratch_types=[
            pltpu.VMEM((_SSUPER * _SCHUNK,), jnp.int32),
            pltpu.VMEM((_SCHUNK,), jnp.int32),
            pltpu.VMEM((_SCHUNK,), jnp.int32),
            pltpu.VMEM((_SSUPER * _SCHUNK,), jnp.int32),
            pltpu.VMEM((_SCHUNK, 128), jnp.float32),
            pltpu.VMEM((_SCHUNK, 128), jnp.float32),
            pltpu.VMEM_SHARED((_N, 128), jnp.float32),
            pltpu.SemaphoreType.DMA,
            pltpu.SemaphoreType.DMA,
            pltpu.SemaphoreType.DMA,
            pltpu.SemaphoreType.DMA,
            pltpu.SemaphoreType.DMA,
            pltpu.SemaphoreType.DMA,
        ],
    )
    out = fn(tab, src, dst, combo)
    return jnp.concatenate([out[0], out[1]], axis=1)


def _enc2_kernel(x_ref, wa_ref, ba_ref, wc_ref, bc_ref, ha_ref, hc_ref):
    r = jax.nn.relu(x_ref[...])
    ha_ref[...] = jnp.dot(r, wa_ref[...], preferred_element_type=jnp.float32) + ba_ref[...]
    hc_ref[...] = jnp.dot(r, wc_ref[...], preferred_element_type=jnp.float32) + bc_ref[...]


def _enc1_kernel(gs_ref, gd_ref, w_ref, b_ref, h_ref):
    r = jax.nn.relu(gs_ref[...].astype(jnp.float32)
                    + gd_ref[...].astype(jnp.float32))
    h_ref[...] = jnp.dot(r, w_ref[...], preferred_element_type=jnp.float32) + b_ref[...]


def _ce_block(logits, lab_col, need_acc):
    m = jnp.max(logits, axis=1, keepdims=True)
    lse = m[:, 0] + jnp.log(jnp.sum(jnp.exp(logits - m), axis=1))
    iota = jax.lax.broadcasted_iota(jnp.int32, logits.shape, 1)
    onehot = iota == lab_col
    picked = jnp.sum(jnp.where(onehot, logits, 0.0), axis=1)
    loss_sum = jnp.sum(lse - picked)
    if need_acc:
        idx = jnp.min(jnp.where(logits == m, iota, jnp.int32(1 << 30)), axis=1)
        acc_sum = jnp.sum((idx == lab_col[:, 0]).astype(jnp.float32))
    else:
        acc_sum = jnp.float32(0.0)
    return loss_sum, acc_sum


def _node_loss_kernel(ha_ref, agga_ref, laba_ref, hc_ref, aggc_ref, labc_ref,
                      wa_ref, ba_ref, wc_ref, bc_ref, epsa_ref, epsc_ref,
                      out_ref):
    i = pl.program_id(0)

    @pl.when(i == 0)
    def _():
        out_ref[...] = jnp.zeros_like(out_ref)

    def one(h, agg, lab, w, bpad, eps, need_acc):
        z = h + eps * h + agg
        logits = jnp.dot(z, w, preferred_element_type=jnp.float32) + bpad
        return _ce_block(logits, lab, need_acc)

    la, acca = one(ha_ref[...], agga_ref[...], laba_ref[...],
                   wa_ref[...], ba_ref[...], epsa_ref[...], True)
    lc, _ = one(hc_ref[...], aggc_ref[...], labc_ref[...],
                wc_ref[...], bc_ref[...], epsc_ref[...], False)
    lane = jax.lax.broadcasted_iota(jnp.int32, (1, 128), 1)
    out_ref[...] += jnp.where(lane == 0, la,
                     jnp.where(lane == 1, lc,
                      jnp.where(lane == 2, acca, 0.0)))


def _edge_head_loss_kernel(gs_ref, gd_ref, lab_ref, agg_ref, we_ref, be_ref,
                           wo_ref, bo_ref, eps_ref, out_ref):
    i = pl.program_id(0)

    @pl.when(i == 0)
    def _():
        out_ref[...] = jnp.zeros_like(out_ref)

    h = jnp.dot(jax.nn.relu(gs_ref[...].astype(jnp.float32)
                            + gd_ref[...].astype(jnp.float32)), we_ref[...],
                preferred_element_type=jnp.float32) + be_ref[...]
    z = h + eps_ref[...] * h + agg_ref[...]
    logits = jnp.dot(z, wo_ref[...], preferred_element_type=jnp.float32) + bo_ref[...]
    ls, _ = _ce_block(logits, lab_ref[...], False)
    lane = jax.lax.broadcasted_iota(jnp.int32, (1, 128), 1)
    out_ref[...] += jnp.where(lane == 0, ls, 0.0)


def _edge_tail_loss_kernel(gs_ref, gd_ref, lab_ref, we_ref, be_ref,
                           wo_ref, bo_ref, eps_ref, out_ref):
    i = pl.program_id(0)

    @pl.when(i == 0)
    def _():
        out_ref[...] = jnp.zeros_like(out_ref)

    h = jnp.dot(jax.nn.relu(gs_ref[...].astype(jnp.float32)
                            + gd_ref[...].astype(jnp.float32)), we_ref[...],
                preferred_element_type=jnp.float32) + be_ref[...]
    z = h + eps_ref[...] * h
    logits = jnp.dot(z, wo_ref[...], preferred_element_type=jnp.float32) + bo_ref[...]
    ls, _ = _ce_block(logits, lab_ref[...], False)
    lane = jax.lax.broadcasted_iota(jnp.int32, (1, 128), 1)
    out_ref[...] += jnp.where(lane == 0, ls, 0.0)


def _pad_out_params(w, b, out_dim):
    wp = jnp.zeros((_HID, 128), jnp.float32).at[:, :out_dim].set(w)
    bp = jnp.full((1, 128), _NEG, jnp.float32).at[0, :out_dim].set(b)
    return wp, bp


def _combo_table(p):
    return (p["E_type"][:, None, :] + p["E_dire"][None, :, :]).reshape(-1, _HID)


_ARB = pltpu.CompilerParams(dimension_semantics=("arbitrary",))


def _run_node_loss(ha, agga, laba, hc, aggc, labc, pa, pc):
    wa, ba = _pad_out_params(pa["W_out"], pa["b_out"], 119)
    wc, bc = _pad_out_params(pc["W_out"], pc["b_out"], 4)
    n_tiles = _N // _NODE_TILE
    grid = (n_tiles,)
    tile = _NODE_TILE
    out = pl.pallas_call(
        _node_loss_kernel,
        grid=grid,
        in_specs=[
            pl.BlockSpec((tile, _HID), lambda i: (i, 0)),
            pl.BlockSpec((tile, _HID), lambda i: (i, 0)),
            pl.BlockSpec((tile, 1), lambda i: (i, 0)),
            pl.BlockSpec((tile, _HID), lambda i: (i, 0)),
            pl.BlockSpec((tile, _HID), lambda i: (i, 0)),
            pl.BlockSpec((tile, 1), lambda i: (i, 0)),
            pl.BlockSpec((_HID, 128), lambda i: (0, 0)),
            pl.BlockSpec((1, 128), lambda i: (0, 0)),
            pl.BlockSpec((_HID, 128), lambda i: (0, 0)),
            pl.BlockSpec((1, 128), lambda i: (0, 0)),
            pl.BlockSpec((1, 1), lambda i: (0, 0)),
            pl.BlockSpec((1, 1), lambda i: (0, 0)),
        ],
        out_specs=pl.BlockSpec((1, 128), lambda i: (0, 0)),
        out_shape=jax.ShapeDtypeStruct((1, 128), jnp.float32),
        compiler_params=_ARB,
    )(ha, agga, laba, hc, aggc, labc, wa, ba, wc, bc,
      jnp.reshape(pa["eps"], (1, 1)), jnp.reshape(pc["eps"], (1, 1)))
    return out[0, 0], out[0, 1], out[0, 2]


def _run_edge_loss(gcat, labe, agg_e, pe):
    wo, bo = _pad_out_params(pe["W_out"], pe["b_out"], 4)
    we = pe["W_enc"]
    be = jnp.reshape(pe["b_enc"], (1, _HID))
    eps = jnp.reshape(pe["eps"], (1, 1))
    head_n = _HEAD_TILES * _EDGE_TILE
    agg_pad = jnp.zeros((head_n, _HID), jnp.float32).at[:_N].set(agg_e)

    et = _E // _EDGE_TILE
    head = pl.pallas_call(
        _edge_head_loss_kernel,
        grid=(_HEAD_TILES,),
        in_specs=[
            pl.BlockSpec((_EDGE_TILE, _IN), lambda i: (i, 0)),
            pl.BlockSpec((_EDGE_TILE, _IN), lambda i: (et + i, 0)),
            pl.BlockSpec((_EDGE_TILE, 1), lambda i: (i, 0)),
            pl.BlockSpec((_EDGE_TILE, _HID), lambda i: (i, 0)),
            pl.BlockSpec((_IN, _HID), lambda i: (0, 0)),
            pl.BlockSpec((1, _HID), lambda i: (0, 0)),
            pl.BlockSpec((_HID, 128), lambda i: (0, 0)),
            pl.BlockSpec((1, 128), lambda i: (0, 0)),
            pl.BlockSpec((1, 1), lambda i: (0, 0)),
        ],
        out_specs=pl.BlockSpec((1, 128), lambda i: (0, 0)),
        out_shape=jax.ShapeDtypeStruct((1, 128), jnp.float32),
        compiler_params=_ARB,
    )(gcat, gcat, labe[:head_n], agg_pad, we, be, wo, bo, eps)

    tail = pl.pallas_call(
        _edge_tail_loss_kernel,
        grid=((_E - head_n) // _EDGE_TILE,),
        in_specs=[
            pl.BlockSpec((_EDGE_TILE, _IN), lambda i: (_HEAD_TILES + i, 0)),
            pl.BlockSpec((_EDGE_TILE, _IN),
                         lambda i: (et + _HEAD_TILES + i, 0)),
            pl.BlockSpec((_EDGE_TILE, 1), lambda i: (i, 0)),
            pl.BlockSpec((_IN, _HID), lambda i: (0, 0)),
            pl.BlockSpec((1, _HID), lambda i: (0, 0)),
            pl.BlockSpec((_HID, 128), lambda i: (0, 0)),
            pl.BlockSpec((1, 128), lambda i: (0, 0)),
            pl.BlockSpec((1, 1), lambda i: (0, 0)),
        ],
        out_specs=pl.BlockSpec((1, 128), lambda i: (0, 0)),
        out_shape=jax.ShapeDtypeStruct((1, 128), jnp.float32),
        compiler_params=_ARB,
    )(gcat, gcat, labe[head_n:], we, be, wo, bo, eps)
    return head[0, 0] + tail[0, 0]


def kernel(node_representation, node_type, node_chiral_type, edge_type,
           edge_dire_type, edge_index, params):
    x = node_representation
    pa, pc, pe = params["atom"], params["chiral"], params["edge"]
    src = edge_index[0]
    dst = edge_index[1]
    combo = edge_type * 3 + edge_dire_type

    ha, hc = pl.pallas_call(
        _enc2_kernel,
        out_shape=[jax.ShapeDtypeStruct((_N, _HID), jnp.float32)] * 2,
    )(x, pa["W_enc"], jnp.reshape(pa["b_enc"], (1, _HID)),
      pc["W_enc"], jnp.reshape(pc["b_enc"], (1, _HID)))

    gcat = _sc_gather(x, jnp.concatenate([src, dst]))

    h_head = pl.pallas_call(
        _enc1_kernel,
        grid=(1,),
        in_specs=[
            pl.BlockSpec((_N, _IN), lambda i: (0, 0)),
            pl.BlockSpec((_N, _IN), lambda i: (_E // _N, 0)),
            pl.BlockSpec((_IN, _HID), lambda i: (0, 0)),
            pl.BlockSpec((1, _HID), lambda i: (0, 0)),
        ],
        out_specs=pl.BlockSpec((_N, _HID), lambda i: (0, 0)),
        out_shape=jax.ShapeDtypeStruct((_N, _HID), jnp.float32),
        compiler_params=_ARB,
    )(gcat, gcat, pe["W_enc"], jnp.reshape(pe["b_enc"], (1, _HID)))

    emb_a = _combo_table(pa)
    emb_c = _combo_table(pc)
    emb_e = _combo_table(pe)
    agg_a = _sc_scatter(ha, emb_a, src, dst, combo)
    agg_c = _sc_scatter(hc, emb_c, src, dst, combo)
    agg_e = _sc_scatter(h_head, emb_e, src, dst, combo)

    laba = node_type[:, None].astype(jnp.int32)
    labc = node_chiral_type[:, None].astype(jnp.int32)
    labe = edge_type[:, None].astype(jnp.int32)

    la_sum, lc_sum, acc_sum = _run_node_loss(ha, agg_a, laba, hc, agg_c, labc,
                                             pa, pc)
    le_sum = _run_edge_loss(gcat, labe, agg_e, pe)

    loss = la_sum / _N + lc_sum / _N + le_sum / _E
    acc = acc_sum / _N
    return (loss, acc)


# speedup vs baseline: 1.2019x; 1.0032x over previous
import functools

import jax
import jax.numpy as jnp
from jax import lax
from jax.experimental import pallas as pl
from jax.experimental.pallas import tpu as pltpu
from jax.experimental.pallas import tpu_sc as plsc

_N = 10000
_E = 320000
_IN = 128
_HID = 256
_NEG = -1e30

_NODE_TILE = 1000
_EDGE_TILE = 3200
_HEAD_TILES = 4


_SC_NW = 32
_GCHUNK = 200


_GCHUNK = 400
_GSUPER = 10


def _sc_gather_body(x_hbm, idx_hbm, out_hbm,
                    idx_v, r0_, r1_, gsem0, gsem1, wsem0, wsem1):
    c = lax.axis_index("c")
    s = lax.axis_index("s")
    wid = s * 2 + c
    per_w = 2 * _E // _SC_NW
    base = wid * per_w
    sup = _GSUPER * _GCHUNK

    rbufs = (r0_, r1_)
    gsems = (gsem0, gsem1)
    wsems = (wsem0, wsem1)

    def super_body(j, carry):
        soff = base + j * sup
        pltpu.sync_copy(idx_hbm.at[pl.ds(soff, sup)], idx_v)

        def fire(r, b):
            sl = pl.ds(r * _GCHUNK, _GCHUNK)
            return pltpu.async_copy(x_hbm.at[idx_v.at[sl]], rbufs[b],
                                    gsems[b])

        pend_w = [None, None]
        pend = fire(0, 0)
        for r in range(_GSUPER):
            b = r & 1
            pend.wait()
            if r < _GSUPER - 1:
                if pend_w[1 - b] is not None:
                    pend_w[1 - b].wait()
                pend = fire(r + 1, 1 - b)
            off = soff + r * _GCHUNK
            pend_w[b] = pltpu.async_copy(
                rbufs[b], out_hbm.at[pl.ds(off, _GCHUNK)], wsems[b])
        for b in range(2):
            if pend_w[b] is not None:
                pend_w[b].wait()
        return carry

    lax.fori_loop(0, per_w // sup, super_body, 0)


def _sc_gather(x, idx_cat):
    mesh = plsc.VectorSubcoreMesh(core_axis_name="c", subcore_axis_name="s")
    fn = pl.kernel(
        _sc_gather_body,
        out_type=jax.ShapeDtypeStruct((2 * _E, _IN), jnp.float32),
        mesh=mesh,
        scratch_types=[
            pltpu.VMEM((_GSUPER * _GCHUNK,), jnp.int32),
            pltpu.VMEM((_GCHUNK, _IN), jnp.float32),
            pltpu.VMEM((_GCHUNK, _IN), jnp.float32),
            pltpu.SemaphoreType.DMA,
            pltpu.SemaphoreType.DMA,
            pltpu.SemaphoreType.DMA,
            pltpu.SemaphoreType.DMA,
        ],
    )
    return fn(x, idx_cat)


_SCHUNK = 160
_ROWS_PER_SUB = 624
_ROW_PIECES = [(0, 160), (160, 160), (320, 160), (480, 144)]


def _relu_table_kernel(h_ref, emb_ref, out_ref):
    out_ref[...] = jax.nn.relu(h_ref[...] + emb_ref[0])[None]


def _relu_table(h_split, emb_split):
    out = pl.pallas_call(
        _relu_table_kernel,
        grid=(24,),
        in_specs=[
            pl.BlockSpec((_N, 128), lambda i: (i // 12, 0)),
            pl.BlockSpec((1, 1, 128), lambda i: (16 * (i // 12) + i % 12, 0, 0)),
        ],
        out_specs=pl.BlockSpec((1, _N, 128), lambda i: (i, 0, 0)),
        out_shape=jax.ShapeDtypeStruct((24, _N, 128), jnp.float32),
        compiler_params=_ARB,
    )(h_split, emb_split[:, None, :])
    return jnp.reshape(out, (24 * _N, 128))


_SSUPER = 25


def _sc_scatter_body(tab_hbm, src_hbm, dst_hbm, combo_hbm, out_hbm,
                     idx_src, dst0, dst1, idx_cmb, rows0, rows1, agg,
                     gsem0, gsem1, dsem0, dsem1, ssem0, ssem1):
    c = lax.axis_index("c")
    s = lax.axis_index("s")
    per_s = _E // 16
    base = s * per_s
    sup = _SSUPER * _SCHUNK

    def zero_buf(e, carry):
        for g in range(_IN // 16):
            rows0[e, pl.ds(g * 16, 16)] = jnp.zeros((16,), jnp.float32)
        return carry
    lax.fori_loop(0, _SCHUNK, zero_buf, 0)
    r0 = s * _ROWS_PER_SUB
    for (o, nrow) in _ROW_PIECES:
        pltpu.sync_copy(rows0.at[pl.ds(0, nrow)],
                        agg.at[pl.ds(r0 + o, nrow)])

    @pl.when(s == 15)
    def _():
        pltpu.sync_copy(rows0.at[pl.ds(0, _N - 16 * _ROWS_PER_SUB)],
                        agg.at[pl.ds(16 * _ROWS_PER_SUB,
                                     _N - 16 * _ROWS_PER_SUB)])
    plsc.subcore_barrier()

    rbufs = (rows0, rows1)
    dbufs = (dst0, dst1)
    gsems = (gsem0, gsem1)
    dsems = (dsem0, dsem1)
    ssems = (ssem0, ssem1)

    def super_body(j, carry):
        soff = base + j * sup
        pltpu.sync_copy(src_hbm.at[pl.ds(soff, sup)], idx_src)
        pltpu.sync_copy(combo_hbm.at[pl.ds(soff, sup)], idx_cmb)

        def gidx(k, carry2):
            sl = pl.ds(k * 16, 16)
            idx_src[sl] = (idx_src[sl] + idx_cmb[sl] * _N
                           + jnp.full((16,), c * 12 * _N, jnp.int32))
            return carry2
        lax.fori_loop(0, sup // 16, gidx, 0, unroll=5)

        def fire(r, b):
            sl = pl.ds(r * _SCHUNK, _SCHUNK)
            g = pltpu.async_copy(tab_hbm.at[idx_src.at[sl]], rbufs[b],
                                 gsems[b])
            d = pltpu.async_copy(dst_hbm.at[pl.ds(soff + r * _SCHUNK,
                                                  _SCHUNK)],
                                 dbufs[b], dsems[b])
            return (g, d)

        pend_s = [None, None]
        pend = fire(0, 0)
        for r in range(_SSUPER):
            b = r & 1
            pend[0].wait()
            pend[1].wait()
            if r < _SSUPER - 1:
                if pend_s[1 - b] is not None:
                    pend_s[1 - b].wait()
                pend = fire(r + 1, 1 - b)
            pend_s[b] = pltpu.async_copy(rbufs[b], agg.at[dbufs[b]],
                                         ssems[b], add=True)
        for b in range(2):
            if pend_s[b] is not None:
                pend_s[b].wait()
        return carry

    lax.fori_loop(0, per_s // sup, super_body, 0)
    plsc.subcore_barrier()

    for (o, nrow) in _ROW_PIECES:
        pltpu.sync_copy(agg.at[pl.ds(r0 + o, nrow)],
                        rows0.at[pl.ds(0, nrow)])
        pltpu.sync_copy(rows0.at[pl.ds(0, nrow)],
                        out_hbm.at[c, pl.ds(r0 + o, nrow)])

    @pl.when(s == 15)
    def _():
        tail = _N - 16 * _ROWS_PER_SUB
        pltpu.sync_copy(agg.at[pl.ds(16 * _ROWS_PER_SUB, tail)],
                        rows0.at[pl.ds(0, tail)])
        pltpu.sync_copy(rows0.at[pl.ds(0, tail)],
                        out_hbm.at[c, pl.ds(16 * _ROWS_PER_SUB, tail)])


def _sc_scatter(h, emb12, src, dst, combo):
    h_split = jnp.concatenate([h[:, :128], h[:, 128:]], axis=0)
    emb_split = jnp.zeros((32, 128), jnp.float32)
    emb_split = emb_split.at[0:12].set(emb12[:, :128])
    emb_split = emb_split.at[16:28].set(emb12[:, 128:])
    tab = _relu_table(h_split, emb_split)
    mesh = plsc.VectorSubcoreMesh(core_axis_name="c", subcore_axis_name="s")
    fn = pl.kernel(
        _sc_scatter_body,
        out_type=jax.ShapeDtypeStruct((2, _N, 128), jnp.float32),
        mesh=mesh,
        scratch_types=[
            pltpu.VMEM((_SSUPER * _SCHUNK,), jnp.int32),
            pltpu.VMEM((_SCHUNK,), jnp.int32),
            pltpu.VMEM((_SCHUNK,), jnp.int32),
            pltpu.VMEM((_SSUPER * _SCHUNK,), jnp.int32),
            pltpu.VMEM((_SCHUNK, 128), jnp.float32),
            pltpu.VMEM((_SCHUNK, 128), jnp.float32),
            pltpu.VMEM_SHARED((_N, 128), jnp.float32),
            pltpu.SemaphoreType.DMA,
            pltpu.SemaphoreType.DMA,
            pltpu.SemaphoreType.DMA,
            pltpu.SemaphoreType.DMA,
            pltpu.SemaphoreType.DMA,
            pltpu.SemaphoreType.DMA,
        ],
    )
    out = fn(tab, src, dst, combo)
    return jnp.concatenate([out[0], out[1]], axis=1)


def _enc2_kernel(x_ref, wa_ref, ba_ref, wc_ref, bc_ref, ha_ref, hc_ref):
    r = jax.nn.relu(x_ref[...])
    ha_ref[...] = jnp.dot(r, wa_ref[...], preferred_element_type=jnp.float32) + ba_ref[...]
    hc_ref[...] = jnp.dot(r, wc_ref[...], preferred_element_type=jnp.float32) + bc_ref[...]


def _enc1_kernel(gs_ref, gd_ref, w_ref, b_ref, h_ref):
    r = jax.nn.relu(gs_ref[...].astype(jnp.float32)
                    + gd_ref[...].astype(jnp.float32))
    h_ref[...] = jnp.dot(r, w_ref[...], preferred_element_type=jnp.float32) + b_ref[...]


def _ce_block(logits, lab_col, need_acc):
    m = jnp.max(logits, axis=1, keepdims=True)
    lse = m[:, 0] + jnp.log(jnp.sum(jnp.exp(logits - m), axis=1))
    iota = jax.lax.broadcasted_iota(jnp.int32, logits.shape, 1)
    onehot = iota == lab_col
    picked = jnp.sum(jnp.where(onehot, logits, 0.0), axis=1)
    loss_sum = jnp.sum(lse - picked)
    if need_acc:
        idx = jnp.min(jnp.where(logits == m, iota, jnp.int32(1 << 30)), axis=1)
        acc_sum = jnp.sum((idx == lab_col[:, 0]).astype(jnp.float32))
    else:
        acc_sum = jnp.float32(0.0)
    return loss_sum, acc_sum


def _node_loss_kernel(ha_ref, agga_ref, laba_ref, hc_ref, aggc_ref, labc_ref,
                      wa_ref, ba_ref, wc_ref, bc_ref, epsa_ref, epsc_ref,
                      out_ref):
    i = pl.program_id(0)

    @pl.when(i == 0)
    def _():
        out_ref[...] = jnp.zeros_like(out_ref)

    def one(h, agg, lab, w, bpad, eps, need_acc):
        z = h + eps * h + agg
        logits = jnp.dot(z, w, preferred_element_type=jnp.float32) + bpad
        return _ce_block(logits, lab, need_acc)

    la, acca = one(ha_ref[...], agga_ref[...], laba_ref[...],
                   wa_ref[...], ba_ref[...], epsa_ref[...], True)
    lc, _ = one(hc_ref[...], aggc_ref[...], labc_ref[...],
                wc_ref[...], bc_ref[...], epsc_ref[...], False)
    lane = jax.lax.broadcasted_iota(jnp.int32, (1, 128), 1)
    out_ref[...] += jnp.where(lane == 0, la,
                     jnp.where(lane == 1, lc,
                      jnp.where(lane == 2, acca, 0.0)))


def _edge_head_loss_kernel(gs_ref, gd_ref, lab_ref, agg_ref, we_ref, be_ref,
                           wo_ref, bo_ref, eps_ref, out_ref):
    i = pl.program_id(0)

    @pl.when(i == 0)
    def _():
        out_ref[...] = jnp.zeros_like(out_ref)

    h = jnp.dot(jax.nn.relu(gs_ref[...].astype(jnp.float32)
                            + gd_ref[...].astype(jnp.float32)), we_ref[...],
                preferred_element_type=jnp.float32) + be_ref[...]
    z = h + eps_ref[...] * h + agg_ref[...]
    logits = jnp.dot(z, wo_ref[...], preferred_element_type=jnp.float32) + bo_ref[...]
    ls, _ = _ce_block(logits, lab_ref[...], False)
    lane = jax.lax.broadcasted_iota(jnp.int32, (1, 128), 1)
    out_ref[...] += jnp.where(lane == 0, ls, 0.0)


def _edge_tail_loss_kernel(gs_ref, gd_ref, lab_ref, we_ref, be_ref,
                           wo_ref, bo_ref, eps_ref, out_ref):
    i = pl.program_id(0)

    @pl.when(i == 0)
    def _():
        out_ref[...] = jnp.zeros_like(out_ref)

    h = jnp.dot(jax.nn.relu(gs_ref[...].astype(jnp.float32)
                            + gd_ref[...].astype(jnp.float32)), we_ref[...],
                preferred_element_type=jnp.float32) + be_ref[...]
    z = h + eps_ref[...] * h
    logits = jnp.dot(z, wo_ref[...], preferred_element_type=jnp.float32) + bo_ref[...]
    ls, _ = _ce_block(logits, lab_ref[...], False)
    lane = jax.lax.broadcasted_iota(jnp.int32, (1, 128), 1)
    out_ref[...] += jnp.where(lane == 0, ls, 0.0)


def _pad_out_params(w, b, out_dim):
    wp = jnp.zeros((_HID, 128), jnp.float32).at[:, :out_dim].set(w)
    bp = jnp.full((1, 128), _NEG, jnp.float32).at[0, :out_dim].set(b)
    return wp, bp


def _combo_table(p):
    return (p["E_type"][:, None, :] + p["E_dire"][None, :, :]).reshape(-1, _HID)


_ARB = pltpu.CompilerParams(dimension_semantics=("arbitrary",))


def _run_node_loss(ha, agga, laba, hc, aggc, labc, pa, pc):
    wa, ba = _pad_out_params(pa["W_out"], pa["b_out"], 119)
    wc, bc = _pad_out_params(pc["W_out"], pc["b_out"], 4)
    n_tiles = _N // _NODE_TILE
    grid = (n_tiles,)
    tile = _NODE_TILE
    out = pl.pallas_call(
        _node_loss_kernel,
        grid=grid,
        in_specs=[
            pl.BlockSpec((tile, _HID), lambda i: (i, 0)),
            pl.BlockSpec((tile, _HID), lambda i: (i, 0)),
            pl.BlockSpec((tile, 1), lambda i: (i, 0)),
            pl.BlockSpec((tile, _HID), lambda i: (i, 0)),
            pl.BlockSpec((tile, _HID), lambda i: (i, 0)),
            pl.BlockSpec((tile, 1), lambda i: (i, 0)),
            pl.BlockSpec((_HID, 128), lambda i: (0, 0)),
            pl.BlockSpec((1, 128), lambda i: (0, 0)),
            pl.BlockSpec((_HID, 128), lambda i: (0, 0)),
            pl.BlockSpec((1, 128), lambda i: (0, 0)),
            pl.BlockSpec((1, 1), lambda i: (0, 0)),
            pl.BlockSpec((1, 1), lambda i: (0, 0)),
        ],
        out_specs=pl.BlockSpec((1, 128), lambda i: (0, 0)),
        out_shape=jax.ShapeDtypeStruct((1, 128), jnp.float32),
        compiler_params=_ARB,
    )(ha, agga, laba, hc, aggc, labc, wa, ba, wc, bc,
      jnp.reshape(pa["eps"], (1, 1)), jnp.reshape(pc["eps"], (1, 1)))
    return out[0, 0], out[0, 1], out[0, 2]


def _run_edge_loss(gcat, labe, agg_e, pe):
    wo, bo = _pad_out_params(pe["W_out"], pe["b_out"], 4)
    we = pe["W_enc"]
    be = jnp.reshape(pe["b_enc"], (1, _HID))
    eps = jnp.reshape(pe["eps"], (1, 1))
    head_n = _HEAD_TILES * _EDGE_TILE
    agg_pad = jnp.zeros((head_n, _HID), jnp.float32).at[:_N].set(agg_e)

    et = _E // _EDGE_TILE
    head = pl.pallas_call(
        _edge_head_loss_kernel,
        grid=(_HEAD_TILES,),
        in_specs=[
            pl.BlockSpec((_EDGE_TILE, _IN), lambda i: (i, 0)),
            pl.BlockSpec((_EDGE_TILE, _IN), lambda i: (et + i, 0)),
            pl.BlockSpec((_EDGE_TILE, 1), lambda i: (i, 0)),
            pl.BlockSpec((_EDGE_TILE, _HID), lambda i: (i, 0)),
            pl.BlockSpec((_IN, _HID), lambda i: (0, 0)),
            pl.BlockSpec((1, _HID), lambda i: (0, 0)),
            pl.BlockSpec((_HID, 128), lambda i: (0, 0)),
            pl.BlockSpec((1, 128), lambda i: (0, 0)),
            pl.BlockSpec((1, 1), lambda i: (0, 0)),
        ],
        out_specs=pl.BlockSpec((1, 128), lambda i: (0, 0)),
        out_shape=jax.ShapeDtypeStruct((1, 128), jnp.float32),
        compiler_params=_ARB,
    )(gcat, gcat, labe[:head_n], agg_pad, we, be, wo, bo, eps)

    tail = pl.pallas_call(
        _edge_tail_loss_kernel,
        grid=((_E - head_n) // _EDGE_TILE,),
        in_specs=[
            pl.BlockSpec((_EDGE_TILE, _IN), lambda i: (_HEAD_TILES + i, 0)),
            pl.BlockSpec((_EDGE_TILE, _IN),
                         lambda i: (et + _HEAD_TILES + i, 0)),
            pl.BlockSpec((_EDGE_TILE, 1), lambda i: (i, 0)),
            pl.BlockSpec((_IN, _HID), lambda i: (0, 0)),
            pl.BlockSpec((1, _HID), lambda i: (0, 0)),
            pl.BlockSpec((_HID, 128), lambda i: (0, 0)),
            pl.BlockSpec((1, 128), lambda i: (0, 0)),
            pl.BlockSpec((1, 1), lambda i: (0, 0)),
        ],
        out_specs=pl.BlockSpec((1, 128), lambda i: (0, 0)),
        out_shape=jax.ShapeDtypeStruct((1, 128), jnp.float32),
        compiler_params=_ARB,
    )(gcat, gcat, labe[head_n:], we, be, wo, bo, eps)
    return head[0, 0] + tail[0, 0]


def kernel(node_representation, node_type, node_chiral_type, edge_type,
           edge_dire_type, edge_index, params):
    x = node_representation
    pa, pc, pe = params["atom"], params["chiral"], params["edge"]
    src = edge_index[0]
    dst = edge_index[1]
    combo = edge_type * 3 + edge_dire_type

    ha, hc = pl.pallas_call(
        _enc2_kernel,
        out_shape=[jax.ShapeDtypeStruct((_N, _HID), jnp.float32)] * 2,
    )(x, pa["W_enc"], jnp.reshape(pa["b_enc"], (1, _HID)),
      pc["W_enc"], jnp.reshape(pc["b_enc"], (1, _HID)))

    gcat = _sc_gather(x, jnp.concatenate([src, dst]))

    h_head = pl.pallas_call(
        _enc1_kernel,
        grid=(1,),
        in_specs=[
            pl.BlockSpec((_N, _IN), lambda i: (0, 0)),
            pl.BlockSpec((_N, _IN), lambda i: (_E // _N, 0)),
            pl.BlockSpec((_IN, _HID), lambda i: (0, 0)),
            pl.BlockSpec((1, _HID), lambda i: (0, 0)),
        ],
        out_specs=pl.BlockSpec((_N, _HID), lambda i: (0, 0)),
        out_shape=jax.ShapeDtypeStruct((_N, _HID), jnp.float32),
        compiler_params=_ARB,
    )(gcat, gcat, pe["W_enc"], jnp.reshape(pe["b_enc"], (1, _HID)))

    emb_a = _combo_table(pa)
    emb_c = _combo_table(pc)
    emb_e = _combo_table(pe)
    agg_a = _sc_scatter(ha, emb_a, src, dst, combo)
    agg_c = _sc_scatter(hc, emb_c, src, dst, combo)
    agg_e = _sc_scatter(h_head, emb_e, src, dst, combo)

    laba = node_type[:, None].astype(jnp.int32)
    labc = node_chiral_type[:, None].astype(jnp.int32)
    labe = edge_type[:, None].astype(jnp.int32)

    la_sum, lc_sum, acc_sum = _run_node_loss(ha, agg_a, laba, hc, agg_c, labc,
                                             pa, pc)
    le_sum = _run_edge_loss(gcat, labe, agg_e, pe)

    loss = la_sum / _N + lc_sum / _N + le_sum / _E
    acc = acc_sum / _N
    return (loss, acc)
